# baseline (speedup 1.0000x reference)
import jax
import jax.numpy as jnp
from jax import lax
from jax.experimental import pallas as pl
from jax.experimental.pallas import tpu as pltpu
from jax.experimental.pallas import tpu_sc as plsc

N = 10000
E = 320000
D = 128
DH = D // 2

NC = 2
NS = 16
NW = NC * NS
CHUNK = 80
E2 = 332800
PAD = E2 - E - N
TRASH = N
NROW = N + 8
EPW = E2 // NW
NCH_W = EPW // CHUNK
EPS = E2 // NS
NCH_S = EPS // CHUNK
STRIPE = N // NS
SUB = 125
HWID = 16

_mesh = plsc.VectorSubcoreMesh(core_axis_name="c", subcore_axis_name="s")
_sc_params = pltpu.CompilerParams(use_tc_tiling_on_sc=False)


def _hist_body(dstc_hbm, hist_hbm, dst_v, ones_v, zero_v, acc_sh):
    cid = lax.axis_index("c")
    sid = lax.axis_index("s")
    wid = cid * NS + sid
    one16 = jnp.ones((16,), jnp.float32)
    zero16 = jnp.zeros((16,), jnp.float32)

    def _init_ones(i, _):
        ones_v[i] = one16
        return 0

    lax.fori_loop(0, CHUNK, _init_ones, 0)

    def _init_zero(i, _):
        zero_v[i] = zero16
        return 0

    lax.fori_loop(0, STRIPE, _init_zero, 0)
    pltpu.sync_copy(zero_v, acc_sh.at[pl.ds(sid * STRIPE, STRIPE)])

    @pl.when(sid == 0)
    def _():
        pltpu.sync_copy(zero_v.at[pl.ds(0, NROW - N)], acc_sh.at[pl.ds(N, NROW - N)])

    plsc.subcore_barrier()

    pltpu.sync_copy(dstc_hbm.at[wid], dst_v)

    def _chunk(j, _):
        pltpu.sync_copy(ones_v, acc_sh.at[dst_v.at[j]], add=True)
        return 0

    lax.fori_loop(0, NCH_W, _chunk, 0)
    plsc.subcore_barrier()
    pltpu.sync_copy(acc_sh.at[pl.ds(sid * STRIPE, STRIPE)], hist_hbm.at[cid, sid])


_hist = pl.kernel(
    _hist_body,
    out_type=jax.ShapeDtypeStruct((NC, NS, STRIPE, HWID), jnp.float32),
    mesh=_mesh,
    scratch_types=[
        pltpu.VMEM((NCH_W, CHUNK), jnp.int32),
        pltpu.VMEM((CHUNK, HWID), jnp.float32),
        pltpu.VMEM((STRIPE, HWID), jnp.float32),
        pltpu.VMEM_SHARED((NROW, HWID), jnp.float32),
    ],
    compiler_params=_sc_params,
)


_NBUF = 5


def _scat_body(src_hbm, dstc_hbm, hs2_hbm, dinv_hbm, b2_hbm, out_hbm,
               src_v, dst_v, r0, r1, r2, r3, r4, zo_v, acc_v,
               idx_v, dinv_v, b_v, acc_sh, s0, s1, s2, s3, s4):
    cid = lax.axis_index("c")
    sid = lax.axis_index("s")
    z16 = jnp.zeros((16,), jnp.float32)
    bufs = (r0, r1, r2, r3, r4)
    sems = (s0, s1, s2, s3, s4)

    def _zrow(i, _):
        for k in range(DH // 16):
            zo_v[i, pl.ds(k * 16, 16)] = z16
        return 0

    lax.fori_loop(0, SUB, _zrow, 0)
    for t in range(STRIPE // SUB):
        pltpu.sync_copy(zo_v, acc_sh.at[pl.ds(sid * STRIPE + t * SUB, SUB)])

    @pl.when(sid == 0)
    def _():
        pltpu.sync_copy(zo_v.at[pl.ds(0, NROW - N)], acc_sh.at[pl.ds(N, NROW - N)])

    pltpu.sync_copy(src_hbm.at[sid], src_v)
    pltpu.sync_copy(dstc_hbm.at[sid], dst_v)
    pltpu.sync_copy(b2_hbm.at[cid], b_v)
    plsc.subcore_barrier()

    hs_c = hs2_hbm.at[cid]
    last = NCH_S - 1

    for l in range(_NBUF - 1):
        pltpu.async_copy(hs_c.at[src_v.at[l]], bufs[l], sems[l])

    def _round(t, _):
        base = _NBUF * t
        for l in range(_NBUF):
            j = base + l
            pltpu.make_async_copy(hs_c.at[src_v.at[j]], bufs[l], sems[l]).wait()
            pltpu.sync_copy(bufs[l], acc_sh.at[dst_v.at[j]], add=True)
            nxt = jnp.minimum(j + _NBUF - 1, last)
            bn = (l + _NBUF - 1) % _NBUF
            pltpu.async_copy(hs_c.at[src_v.at[nxt]], bufs[bn], sems[bn])
        return 0

    lax.fori_loop(0, NCH_S // _NBUF, _round, 0)
    for l in range(_NBUF - 1):
        pltpu.make_async_copy(hs_c.at[src_v.at[last]], bufs[l], sems[l]).wait()
    plsc.subcore_barrier()

    bk = tuple(b_v[pl.ds(k * 16, 16)] for k in range(DH // 16))
    row0 = sid * STRIPE
    iota16 = lax.iota(jnp.int32, 16)

    def _sub(t, _):
        r0_ = row0 + t * SUB
        for off in (0, 16, 32, 48, 64, 80, 96, SUB - 16):
            idx_v[pl.ds(off, 16)] = iota16 + (r0_ + off)
        pltpu.async_copy(dinv_hbm.at[idx_v], dinv_v, s0).wait()
        pltpu.sync_copy(acc_sh.at[pl.ds(r0_, SUB)], acc_v)

        def _row(r, _2):
            dv = dinv_v[r]
            for k in range(DH // 16):
                sl = pl.ds(k * 16, 16)
                zo_v[r, sl] = acc_v[r, sl] * dv + bk[k]
            return 0

        lax.fori_loop(0, SUB, _row, 0)
        pltpu.sync_copy(zo_v, out_hbm.at[cid, sid, pl.ds(t * SUB, SUB)])
        return 0

    lax.fori_loop(0, STRIPE // SUB, _sub, 0)


_scatter = pl.kernel(
    _scat_body,
    out_type=jax.ShapeDtypeStruct((NC, NS, STRIPE, DH), jnp.float32),
    mesh=_mesh,
    scratch_types=[
        pltpu.VMEM((NCH_S, CHUNK), jnp.int32),
        pltpu.VMEM((NCH_S, CHUNK), jnp.int32),
    ] + [pltpu.VMEM((CHUNK, DH), jnp.float32)] * _NBUF + [
        pltpu.VMEM((SUB, DH), jnp.float32),
        pltpu.VMEM((SUB, DH), jnp.float32),
        pltpu.VMEM((SUB,), jnp.int32),
        pltpu.VMEM((SUB, HWID), jnp.float32),
        pltpu.VMEM((DH,), jnp.float32),
        pltpu.VMEM_SHARED((NROW, DH), jnp.float32),
    ] + [pltpu.SemaphoreType.DMA] * _NBUF,
    compiler_params=_sc_params,
)


_RB = 400
_NB = N // _RB


def _mm_body(hist_ref, emb_ref, w_ref, hs2_ref, dinv_ref):
    deg = hist_ref[0] + hist_ref[1]
    dinv = lax.rsqrt(deg)
    h = jnp.dot(emb_ref[...], w_ref[...], preferred_element_type=jnp.float32)
    hs = h * dinv[:, :1]
    hs2_ref[0] = hs[:, :DH]
    hs2_ref[1] = hs[:, DH:]
    dinv_ref[...] = dinv


def _mm(hist, emb, w):
    return pl.pallas_call(
        _mm_body,
        grid=(_NB,),
        in_specs=[
            pl.BlockSpec((NC, _RB, HWID), lambda i: (0, i, 0)),
            pl.BlockSpec((_RB, D), lambda i: (i, 0)),
            pl.BlockSpec((D, D), lambda i: (0, 0)),
        ],
        out_specs=[
            pl.BlockSpec((NC, _RB, DH), lambda i: (0, i, 0)),
            pl.BlockSpec((_RB, HWID), lambda i: (i, 0)),
        ],
        out_shape=[
            jax.ShapeDtypeStruct((NC, N, DH), jnp.float32),
            jax.ShapeDtypeStruct((N, HWID), jnp.float32),
        ],
    )(hist, emb, w)


def kernel(embedding, top_to_bottom_edge_index, W, b):
    ei = top_to_bottom_edge_index.astype(jnp.int32)
    loop = jnp.arange(N, dtype=jnp.int32)
    pad_src = jnp.concatenate([loop, jnp.zeros((PAD,), jnp.int32)])
    pad_dst = jnp.concatenate([loop, jnp.full((PAD,), TRASH, jnp.int32)])
    ei2 = jnp.concatenate([ei, jnp.stack([pad_src, pad_dst])], axis=1)
    src_s = ei2[0].reshape(NS, NCH_S, CHUNK)
    dst_s = ei2[1].reshape(NS, NCH_S, CHUNK)
    dst_w = ei2[1].reshape(NW, NCH_W, CHUNK)
    hist = _hist(dst_w).reshape(NC, N, HWID)
    hs2, dinv = _mm(hist, embedding, W)
    b2 = jnp.stack([b[:DH], b[DH:]])
    out4 = _scatter(src_s, dst_s, hs2, dinv, b2)
    return out4.reshape(NC, N, DH).transpose(1, 0, 2).reshape(N, D)


# speedup vs baseline: 1.4514x; 1.4514x over previous
import jax
import jax.numpy as jnp
from jax import lax
from jax.experimental import pallas as pl
from jax.experimental.pallas import tpu as pltpu
from jax.experimental.pallas import tpu_sc as plsc

N = 10000
E = 320000
D = 128
DH = D // 2

NC = 2
NS = 16
NW = NC * NS
CHUNK = 80
EPW = E // NW
NCH_W = EPW // CHUNK
EPS = E // NS
NCH_S = EPS // CHUNK
STRIPE = N // NS
HWID = 16

_mesh = plsc.VectorSubcoreMesh(core_axis_name="c", subcore_axis_name="s")
_sc_params = pltpu.CompilerParams(use_tc_tiling_on_sc=False)


def _hist_body(dstc_hbm, hist_hbm, dst_v, ones_v, zero_v, acc_sh):
    cid = lax.axis_index("c")
    sid = lax.axis_index("s")
    wid = cid * NS + sid
    one16 = jnp.ones((16,), jnp.float32)
    zero16 = jnp.zeros((16,), jnp.float32)

    def _init_ones(i, _):
        ones_v[i] = one16
        return 0

    lax.fori_loop(0, CHUNK, _init_ones, 0)

    def _init_zero(i, _):
        zero_v[i] = zero16
        return 0

    lax.fori_loop(0, STRIPE, _init_zero, 0)
    pltpu.sync_copy(zero_v, acc_sh.at[pl.ds(sid * STRIPE, STRIPE)])
    plsc.subcore_barrier()

    pltpu.sync_copy(dstc_hbm.at[wid], dst_v)

    def _chunk(j, _):
        pltpu.sync_copy(ones_v, acc_sh.at[dst_v.at[j]], add=True)
        return 0

    lax.fori_loop(0, NCH_W, _chunk, 0)
    plsc.subcore_barrier()
    pltpu.sync_copy(acc_sh.at[pl.ds(sid * STRIPE, STRIPE)], hist_hbm.at[cid, sid])


_hist = pl.kernel(
    _hist_body,
    out_type=jax.ShapeDtypeStruct((NC, NS, STRIPE, HWID), jnp.float32),
    mesh=_mesh,
    scratch_types=[
        pltpu.VMEM((NCH_W, CHUNK), jnp.int32),
        pltpu.VMEM((CHUNK, HWID), jnp.float32),
        pltpu.VMEM((STRIPE, HWID), jnp.float32),
        pltpu.VMEM_SHARED((N, HWID), jnp.float32),
    ],
    compiler_params=_sc_params,
)


_NBUF = 5


def _scat_body(src_hbm, dstc_hbm, hs2_hbm, out_hbm,
               src_v, dst_v, r0, r1, r2, r3, r4, zero_v, acc_sh,
               s0, s1, s2, s3, s4):
    cid = lax.axis_index("c")
    sid = lax.axis_index("s")
    z16 = jnp.zeros((16,), jnp.float32)
    bufs = (r0, r1, r2, r3, r4)
    sems = (s0, s1, s2, s3, s4)

    def _zrow(i, _):
        for k in range(DH // 16):
            zero_v[i, pl.ds(k * 16, 16)] = z16
        return 0

    lax.fori_loop(0, 125, _zrow, 0)
    for t in range(STRIPE // 125):
        pltpu.sync_copy(zero_v, acc_sh.at[pl.ds(sid * STRIPE + t * 125, 125)])

    pltpu.sync_copy(src_hbm.at[sid], src_v)
    pltpu.sync_copy(dstc_hbm.at[sid], dst_v)
    plsc.subcore_barrier()

    hs_c = hs2_hbm.at[cid]
    last = NCH_S - 1

    for l in range(_NBUF - 1):
        pltpu.async_copy(hs_c.at[src_v.at[l]], bufs[l], sems[l])

    def _round(t, _):
        base = _NBUF * t
        for l in range(_NBUF):
            j = base + l
            pltpu.make_async_copy(hs_c.at[src_v.at[j]], bufs[l], sems[l]).wait()
            pltpu.sync_copy(bufs[l], acc_sh.at[dst_v.at[j]], add=True)
            nxt = jnp.minimum(j + _NBUF - 1, last)
            bn = (l + _NBUF - 1) % _NBUF
            pltpu.async_copy(hs_c.at[src_v.at[nxt]], bufs[bn], sems[bn])
        return 0

    lax.fori_loop(0, NCH_S // _NBUF, _round, 0)
    for l in range(_NBUF - 1):
        pltpu.make_async_copy(hs_c.at[src_v.at[last]], bufs[l], sems[l]).wait()
    plsc.subcore_barrier()
    pltpu.sync_copy(acc_sh.at[pl.ds(sid * STRIPE, STRIPE)], out_hbm.at[cid, sid])


_scatter = pl.kernel(
    _scat_body,
    out_type=jax.ShapeDtypeStruct((NC, NS, STRIPE, DH), jnp.float32),
    mesh=_mesh,
    scratch_types=[
        pltpu.VMEM((NCH_S, CHUNK), jnp.int32),
        pltpu.VMEM((NCH_S, CHUNK), jnp.int32),
    ] + [pltpu.VMEM((CHUNK, DH), jnp.float32)] * _NBUF + [
        pltpu.VMEM((125, DH), jnp.float32),
        pltpu.VMEM_SHARED((N, DH), jnp.float32),
    ] + [pltpu.SemaphoreType.DMA] * _NBUF,
    compiler_params=_sc_params,
)


_RB = 400
_NB = N // _RB


def _mm_body(hist_ref, emb_ref, w_ref, hs2_ref):
    deg = hist_ref[0] + hist_ref[1] + 1.0
    dinv = lax.rsqrt(deg)[:, :1]
    h = jnp.dot(emb_ref[...], w_ref[...], preferred_element_type=jnp.float32)
    hs = h * dinv
    hs2_ref[0] = hs[:, :DH]
    hs2_ref[1] = hs[:, DH:]


def _mm(hist, emb, w):
    return pl.pallas_call(
        _mm_body,
        grid=(_NB,),
        in_specs=[
            pl.BlockSpec((NC, _RB, HWID), lambda i: (0, i, 0)),
            pl.BlockSpec((_RB, D), lambda i: (i, 0)),
            pl.BlockSpec((D, D), lambda i: (0, 0)),
        ],
        out_specs=pl.BlockSpec((NC, _RB, DH), lambda i: (0, i, 0)),
        out_shape=jax.ShapeDtypeStruct((NC, N, DH), jnp.float32),
    )(hist, emb, w)


def _ep_body(acc_ref, hs2_ref, hist_ref, b_ref, out_ref):
    deg = hist_ref[0] + hist_ref[1] + 1.0
    dinv = lax.rsqrt(deg)[:, :1]
    s0 = (acc_ref[0] + hs2_ref[0]) * dinv
    s1 = (acc_ref[1] + hs2_ref[1]) * dinv
    out_ref[...] = jnp.concatenate([s0, s1], axis=1) + b_ref[...]


def _ep(acc, hs2, hist, b2):
    return pl.pallas_call(
        _ep_body,
        grid=(_NB,),
        in_specs=[
            pl.BlockSpec((NC, _RB, DH), lambda i: (0, i, 0)),
            pl.BlockSpec((NC, _RB, DH), lambda i: (0, i, 0)),
            pl.BlockSpec((NC, _RB, HWID), lambda i: (0, i, 0)),
            pl.BlockSpec((1, D), lambda i: (0, 0)),
        ],
        out_specs=pl.BlockSpec((_RB, D), lambda i: (i, 0)),
        out_shape=jax.ShapeDtypeStruct((N, D), jnp.float32),
    )(acc, hs2, hist, b2)


def kernel(embedding, top_to_bottom_edge_index, W, b):
    ei = top_to_bottom_edge_index.astype(jnp.int32)
    src_s = ei[0].reshape(NS, NCH_S, CHUNK)
    dst_s = ei[1].reshape(NS, NCH_S, CHUNK)
    dst_w = ei[1].reshape(NW, NCH_W, CHUNK)
    hist = _hist(dst_w).reshape(NC, N, HWID)
    hs2 = _mm(hist, embedding, W)
    acc = _scatter(src_s, dst_s, hs2).reshape(NC, N, DH)
    return _ep(acc, hs2, hist, b.reshape(1, D))


# speedup vs baseline: 1.4994x; 1.0331x over previous
import jax
import jax.numpy as jnp
from jax import lax
from jax.experimental import pallas as pl
from jax.experimental.pallas import tpu as pltpu
from jax.experimental.pallas import tpu_sc as plsc

N = 10000
E = 320000
D = 128
DH = D // 2

NC = 2
NS = 16
NW = NC * NS
CHUNK = 80
EPW = E // NW
NCH_W = EPW // CHUNK
EPS = E // NS
NCH_S = EPS // CHUNK
STRIPE = N // NS
HWID = 16

_mesh = plsc.VectorSubcoreMesh(core_axis_name="c", subcore_axis_name="s")
_sc_params = pltpu.CompilerParams(use_tc_tiling_on_sc=False)


def _hist_body(dstc_hbm, hist_hbm, dst_v, ones_v, zero_v, acc_sh, hsem):
    cid = lax.axis_index("c")
    sid = lax.axis_index("s")
    wid = cid * NS + sid
    one16 = jnp.ones((16,), jnp.float32)
    zero16 = jnp.zeros((16,), jnp.float32)

    def _init_ones(i, _):
        ones_v[i] = one16
        return 0

    lax.fori_loop(0, CHUNK, _init_ones, 0)

    def _init_zero(i, _):
        zero_v[i] = zero16
        return 0

    lax.fori_loop(0, STRIPE, _init_zero, 0)
    pltpu.sync_copy(zero_v, acc_sh.at[pl.ds(sid * STRIPE, STRIPE)])
    plsc.subcore_barrier()

    pltpu.sync_copy(dstc_hbm.at[wid], dst_v)

    def _chunk(j, _):
        pltpu.async_copy(ones_v, acc_sh.at[dst_v.at[j]], hsem, add=True)
        return 0

    lax.fori_loop(0, NCH_W, _chunk, 0)

    def _drain(j, _):
        pltpu.make_async_copy(ones_v, acc_sh.at[dst_v.at[j]], hsem).wait()
        return 0

    lax.fori_loop(0, NCH_W, _drain, 0)
    plsc.subcore_barrier()
    pltpu.sync_copy(acc_sh.at[pl.ds(sid * STRIPE, STRIPE)], hist_hbm.at[cid, sid])


_hist = pl.kernel(
    _hist_body,
    out_type=jax.ShapeDtypeStruct((NC, NS, STRIPE, HWID), jnp.float32),
    mesh=_mesh,
    scratch_types=[
        pltpu.VMEM((NCH_W, CHUNK), jnp.int32),
        pltpu.VMEM((CHUNK, HWID), jnp.float32),
        pltpu.VMEM((STRIPE, HWID), jnp.float32),
        pltpu.VMEM_SHARED((N, HWID), jnp.float32),
        pltpu.SemaphoreType.DMA,
    ],
    compiler_params=_sc_params,
)


_NBUF = 5


def _scat_body(src_hbm, dstc_hbm, hs2_hbm, out_hbm,
               src_v, dst_v, r0, r1, r2, r3, r4, zero_v, acc_sh,
               s0, s1, s2, s3, s4):
    cid = lax.axis_index("c")
    sid = lax.axis_index("s")
    z16 = jnp.zeros((16,), jnp.float32)
    bufs = (r0, r1, r2, r3, r4)
    sems = (s0, s1, s2, s3, s4)

    def _zrow(i, _):
        for k in range(DH // 16):
            zero_v[i, pl.ds(k * 16, 16)] = z16
        return 0

    lax.fori_loop(0, 125, _zrow, 0)
    for t in range(STRIPE // 125):
        pltpu.sync_copy(zero_v, acc_sh.at[pl.ds(sid * STRIPE + t * 125, 125)])

    pltpu.sync_copy(src_hbm.at[sid], src_v)
    pltpu.sync_copy(dstc_hbm.at[sid], dst_v)
    plsc.subcore_barrier()

    hs_c = hs2_hbm.at[cid]
    last = NCH_S - 1

    for l in range(_NBUF - 1):
        pltpu.async_copy(hs_c.at[src_v.at[l]], bufs[l], sems[l])

    def _round(t, _):
        base = _NBUF * t
        for l in range(_NBUF):
            j = base + l
            pltpu.make_async_copy(hs_c.at[src_v.at[j]], bufs[l], sems[l]).wait()
            pltpu.sync_copy(bufs[l], acc_sh.at[dst_v.at[j]], add=True)
            nxt = jnp.minimum(j + _NBUF - 1, last)
            bn = (l + _NBUF - 1) % _NBUF
            pltpu.async_copy(hs_c.at[src_v.at[nxt]], bufs[bn], sems[bn])
        return 0

    lax.fori_loop(0, NCH_S // _NBUF, _round, 0)
    for l in range(_NBUF - 1):
        pltpu.make_async_copy(hs_c.at[src_v.at[last]], bufs[l], sems[l]).wait()
    plsc.subcore_barrier()
    pltpu.sync_copy(acc_sh.at[pl.ds(sid * STRIPE, STRIPE)], out_hbm.at[cid, sid])


_scatter = pl.kernel(
    _scat_body,
    out_type=jax.ShapeDtypeStruct((NC, NS, STRIPE, DH), jnp.float32),
    mesh=_mesh,
    scratch_types=[
        pltpu.VMEM((NCH_S, CHUNK), jnp.int32),
        pltpu.VMEM((NCH_S, CHUNK), jnp.int32),
    ] + [pltpu.VMEM((CHUNK, DH), jnp.float32)] * _NBUF + [
        pltpu.VMEM((125, DH), jnp.float32),
        pltpu.VMEM_SHARED((N, DH), jnp.float32),
    ] + [pltpu.SemaphoreType.DMA] * _NBUF,
    compiler_params=_sc_params,
)


_RB = 400
_NB = N // _RB


def _mm_body(hist_ref, emb_ref, w_ref, hs2_ref):
    deg = hist_ref[0] + hist_ref[1] + 1.0
    dinv = lax.rsqrt(deg)[:, :1]
    h = jnp.dot(emb_ref[...], w_ref[...], preferred_element_type=jnp.float32)
    hs = h * dinv
    hs2_ref[0] = hs[:, :DH]
    hs2_ref[1] = hs[:, DH:]


def _mm(hist, emb, w):
    return pl.pallas_call(
        _mm_body,
        grid=(_NB,),
        in_specs=[
            pl.BlockSpec((NC, _RB, HWID), lambda i: (0, i, 0)),
            pl.BlockSpec((_RB, D), lambda i: (i, 0)),
            pl.BlockSpec((D, D), lambda i: (0, 0)),
        ],
        out_specs=pl.BlockSpec((NC, _RB, DH), lambda i: (0, i, 0)),
        out_shape=jax.ShapeDtypeStruct((NC, N, DH), jnp.float32),
    )(hist, emb, w)


def _ep_body(acc_ref, hs2_ref, hist_ref, b_ref, out_ref):
    deg = hist_ref[0] + hist_ref[1] + 1.0
    dinv = lax.rsqrt(deg)[:, :1]
    s0 = (acc_ref[0] + hs2_ref[0]) * dinv
    s1 = (acc_ref[1] + hs2_ref[1]) * dinv
    out_ref[...] = jnp.concatenate([s0, s1], axis=1) + b_ref[...]


def _ep(acc, hs2, hist, b2):
    return pl.pallas_call(
        _ep_body,
        grid=(_NB,),
        in_specs=[
            pl.BlockSpec((NC, _RB, DH), lambda i: (0, i, 0)),
            pl.BlockSpec((NC, _RB, DH), lambda i: (0, i, 0)),
            pl.BlockSpec((NC, _RB, HWID), lambda i: (0, i, 0)),
            pl.BlockSpec((1, D), lambda i: (0, 0)),
        ],
        out_specs=pl.BlockSpec((_RB, D), lambda i: (i, 0)),
        out_shape=jax.ShapeDtypeStruct((N, D), jnp.float32),
    )(acc, hs2, hist, b2)


def kernel(embedding, top_to_bottom_edge_index, W, b):
    ei = top_to_bottom_edge_index.astype(jnp.int32)
    src_s = ei[0].reshape(NS, NCH_S, CHUNK)
    dst_s = ei[1].reshape(NS, NCH_S, CHUNK)
    dst_w = ei[1].reshape(NW, NCH_W, CHUNK)
    hist = _hist(dst_w).reshape(NC, N, HWID)
    hs2 = _mm(hist, embedding, W)
    acc = _scatter(src_s, dst_s, hs2).reshape(NC, N, DH)
    return _ep(acc, hs2, hist, b.reshape(1, D))
